# baseline (speedup 1.0000x reference)
import dataclasses

import jax
import jax.numpy as jnp
from jax import lax
from jax.experimental import pallas as pl
from jax.experimental.pallas import tpu as pltpu
from jax.experimental.pallas import tpu_sc as plsc

N_USERS = 25000
N_ITEMS = 25000
N = N_USERS + N_ITEMS
E = 800000
EMB = 64
HALF = 32
ATT = 32
C = 3

NC = 2
NS = 16
CHUNK = 128
SB = 40
NSB = 10
ROWS_PER_SUB = SB * NSB
EROWS = NS * ROWS_PER_SUB
EPAD = EROWS * CHUNK
ACC_PER_SUB = 3136
NPAD = NS * ACC_PER_SUB
G = 4
NGSB = SB // G
ZROWS = 32

_F32 = jnp.float32
_BF16 = jnp.bfloat16


def _leaky(x):
    return jnp.where(x >= 0, x, 0.3 * x)


def _sc_spmm_body(xall, src_hbm, dst_hbm, val_hbm, yall,
                  accum, sidx, didx, valv, rowsb, zbuf, gsem, ssem, isem):
    c = lax.axis_index("c")
    s = lax.axis_index("s")
    coff16 = jnp.full((16,), c * (C * N), jnp.int32)

    zero32 = jnp.zeros((32,), _BF16)

    @pl.loop(0, ZROWS)
    def _(r):
        zbuf[r, pl.ds(0, 32)] = zero32

    def zero_accum():
        @pl.loop(0, ACC_PER_SUB // ZROWS)
        def _(t):
            pltpu.sync_copy(
                zbuf, accum.at[pl.ds(s * ACC_PER_SUB + t * ZROWS, ZROWS)])

    zero_accum()
    plsc.subcore_barrier()

    gd = lax.GatherDimensionNumbers(
        offset_dims=(), collapsed_slice_dims=(0,), start_index_map=(0,))

    def lane_bcast(v16, k):
        return lax.gather(
            v16, jnp.full((16, 1), k, jnp.int32), gd, (1,),
            mode=lax.GatherScatterMode.PROMISE_IN_BOUNDS)

    @pl.loop(0, C)
    def _(i):
        base = i * EROWS + s * ROWS_PER_SUB

        def fire_gathers(g, slot):
            for j in range(G):
                pltpu.async_copy(
                    xall.at[sidx.at[g * G + j]],
                    rowsb.at[slot * G + j], gsem.at[slot])

        def drain(sem_slot):
            for j in range(G):
                pltpu.make_async_copy(
                    xall.at[pl.ds(0, CHUNK)], rowsb.at[j], sem_slot).wait()

        def scale(g, slot):
            for j in range(G):
                r = slot * G + j
                ir = g * G + j

                @pl.loop(0, CHUNK, step=16)
                def _(e0):
                    v16 = valv[ir, pl.ds(e0, 16)]
                    for k in range(16):
                        vb = lane_bcast(v16, k)
                        vb2 = plsc.pack(
                            vb, vb, format=plsc.PackFormat.INTERLEAVED)
                        e = e0 + k
                        rowsb[r, e, pl.ds(0, 32)] = (
                            rowsb[r, e, pl.ds(0, 32)] * vb2)

        def fire_scatters(g, slot):
            for j in range(G):
                pltpu.async_copy(
                    rowsb.at[slot * G + j], accum.at[didx.at[g * G + j]],
                    ssem.at[slot], add=True)

        def group_step(g, slot, first, last):
            other = 1 - slot
            drain(gsem.at[slot])
            scale(g, slot)
            if not first:
                drain(ssem.at[other])
            fire_scatters(g, slot)
            if not last:
                fire_gathers(g + 1, other)

        @pl.loop(0, NSB)
        def _(b):
            sb = base + b * SB
            d1 = pltpu.async_copy(src_hbm.at[pl.ds(sb, SB)], sidx, isem)
            d2 = pltpu.async_copy(dst_hbm.at[pl.ds(sb, SB)], didx, isem)
            d3 = pltpu.async_copy(val_hbm.at[pl.ds(sb, SB)], valv, isem)
            d1.wait()
            d2.wait()
            d3.wait()

            @pl.loop(0, SB)
            def _(r):
                @pl.loop(0, CHUNK, step=16)
                def _(k):
                    sidx[r, pl.ds(k, 16)] = sidx[r, pl.ds(k, 16)] + coff16

            fire_gathers(0, 0)
            group_step(0, 0, True, False)

            @pl.loop(1, NGSB - 1, step=2)
            def _(g):
                group_step(g, 1, False, False)
                group_step(g + 1, 0, False, False)

            group_step(NGSB - 1, 1, False, True)
            drain(ssem.at[1])

        plsc.subcore_barrier()
        off = (c * C + i) * NPAD + s * ACC_PER_SUB
        pltpu.sync_copy(
            accum.at[pl.ds(s * ACC_PER_SUB, ACC_PER_SUB)],
            yall.at[pl.ds(off, ACC_PER_SUB)])
        zero_accum()
        plsc.subcore_barrier()


def _sc_compiler_params():
    cp = pltpu.CompilerParams()
    fields = pltpu.CompilerParams.__dataclass_fields__
    if "needs_layout_passes" in fields:
        cp = dataclasses.replace(cp, needs_layout_passes=False)
    if "use_tc_tiling_on_sc" in fields:
        cp = dataclasses.replace(cp, use_tc_tiling_on_sc=False)
    return cp


def _sc_spmm3(xall, src_cat, dst_cat, val_cat):
    mesh = plsc.VectorSubcoreMesh(
        core_axis_name="c", subcore_axis_name="s",
        num_cores=NC, num_subcores=NS)
    fn = pl.kernel(
        _sc_spmm_body,
        out_type=jax.ShapeDtypeStruct((NC * C * NPAD, HALF), _BF16),
        mesh=mesh,
        scratch_types=[
            pltpu.VMEM_SHARED((NPAD, HALF), _BF16),
            pltpu.VMEM((SB, CHUNK), jnp.int32),
            pltpu.VMEM((SB, CHUNK), jnp.int32),
            pltpu.VMEM((SB, CHUNK), _F32),
            pltpu.VMEM((2 * G, CHUNK, HALF), _BF16),
            pltpu.VMEM((ZROWS, HALF), _BF16),
            pltpu.SemaphoreType.DMA((2,)),
            pltpu.SemaphoreType.DMA((2,)),
            pltpu.SemaphoreType.DMA,
        ],
        name="sc_spmm3",
        compiler_params=_sc_compiler_params(),
    )
    return fn(xall, src_cat, dst_cat, val_cat)


_BLK = 2000
_GRID = N // _BLK


def _attention_mix(g, s1v, s2v):
    t = [jnp.tanh(jnp.dot(g[cc].astype(_BF16), s1v,
                          preferred_element_type=_F32))
         for cc in range(C)]
    outs = []
    for i in range(C):
        s2row = s2v[i][None, :]
        l = [jnp.sum(t[cc][:, 32 * i:32 * i + 32] * s2row,
                     axis=1, keepdims=True) for cc in range(C)]
        m = jnp.maximum(jnp.maximum(l[0], l[1]), l[2])
        w = [jnp.exp(x - m) for x in l]
        tot = w[0] + w[1] + w[2]
        outs.append(
            _leaky((w[0] * g[0] + w[1] * g[1] + w[2] * g[2]) / tot))
    return outs


def _gcn_transform(y3, wgv, wgcv, cev):
    g = []
    for cc in range(C):
        e = jnp.concatenate([y3[cc], y3[C + cc]], axis=1)
        h = jnp.dot(e, wgv, preferred_element_type=_F32)
        h = (h * cev[cc][None, :]).astype(_BF16)
        h = _leaky(jnp.dot(h, wgcv, preferred_element_type=_F32))
        g.append(h)
    return g


def _make_tc_layer(final):
    def body(*refs):
        if final:
            y3, x1, p00, p01, p02, wg, wgc, ce, s1c, s2c = refs[:10]
            outs = refs[10:]
            p0 = (p00, p01, p02)
        else:
            y3, wg, wgc, ce, s1c, s2c, wr0, wr1 = refs[:8]
            outs = refs[8:]

        g = _gcn_transform(y3, wg[...], wgc[...], ce[...])
        mix = _attention_mix(g, s1c[...], s2c[...])

        for i in range(C):
            if final:
                p1 = jnp.concatenate(
                    [x1[i], x1[C + i]], axis=1).astype(_F32)
                outs[0][:, i, :] = (p0[i][...] + p1 + mix[i]) * (1.0 / 3.0)
            else:
                ob = mix[i].astype(_BF16)
                outs[0][i] = ob[:, :HALF]
                outs[0][C + i] = ob[:, HALF:]

        if not final:
            @pl.when(pl.program_id(0) == 0)
            def _():
                c0 = ce[...].astype(_F32)
                c1 = _leaky(jnp.dot(c0, wr0[...],
                                    preferred_element_type=_F32))
                c2 = _leaky(jnp.dot(c1, wr1[...],
                                    preferred_element_type=_F32))
                outs[1][...] = c1
                outs[2][...] = (c0 + c1 + c2) * (1.0 / 3.0)

    row_spec = lambda w: pl.BlockSpec((_BLK, w), lambda b: (b, 0))
    wt_spec = lambda a, b_: pl.BlockSpec((a, b_), lambda b: (0, 0))
    y3_spec = pl.BlockSpec((2 * C, _BLK, HALF), lambda b: (0, b, 0))

    wt_specs = [wt_spec(EMB, EMB), wt_spec(EMB, EMB),
                wt_spec(8, EMB), wt_spec(EMB, 96), wt_spec(8, 32)]
    if final:
        in_specs = [y3_spec, y3_spec] + [row_spec(EMB)] * 3 + wt_specs
        out_specs = [pl.BlockSpec((_BLK, C, EMB), lambda b: (b, 0, 0))]
        out_shape = [jax.ShapeDtypeStruct((N, C, EMB), _F32)]
    else:
        in_specs = [y3_spec] + wt_specs + [wt_spec(EMB, EMB)] * 2
        out_specs = [
            pl.BlockSpec((2 * C, _BLK, HALF), lambda b: (0, b, 0)),
            wt_spec(8, EMB), wt_spec(8, EMB)]
        out_shape = [
            jax.ShapeDtypeStruct((2 * C, N, HALF), _BF16),
            jax.ShapeDtypeStruct((8, EMB), _F32),
            jax.ShapeDtypeStruct((8, EMB), _F32)]

    return pl.pallas_call(
        body,
        grid=(_GRID,),
        in_specs=in_specs,
        out_specs=out_specs,
        out_shape=out_shape,
    )


def kernel(adj_idx_0, adj_val_0, adj_idx_1, adj_val_1, adj_idx_2, adj_val_2,
           user_embedding, item_embedding, criterion_embedding, w_gcn,
           W_gc_0, W_gc_1, W_rel_0, W_rel_1, trans_s1, trans_s2):
    pad = EPAD - E
    adj = ((adj_idx_0, adj_val_0), (adj_idx_1, adj_val_1),
           (adj_idx_2, adj_val_2))
    src_cat = jnp.concatenate(
        [(jnp.pad(ai[1], (0, pad)) + i * N).reshape(EROWS, CHUNK)
         for i, (ai, _) in enumerate(adj)], axis=0)
    dst_cat = jnp.concatenate(
        [jnp.pad(ai[0], (0, pad)).reshape(EROWS, CHUNK)
         for ai, _ in adj], axis=0)
    val_cat = jnp.concatenate(
        [jnp.pad(av, (0, pad)).reshape(EROWS, CHUNK)
         for _, av in adj], axis=0)

    pre = jnp.concatenate([user_embedding, item_embedding], axis=0)
    p0 = [pre[:, i, :] for i in range(C)]
    xall0 = jnp.concatenate(
        [p0[i][:, :HALF] for i in range(C)]
        + [p0[i][:, HALF:] for i in range(C)], axis=0).astype(_BF16)

    ce0 = jnp.pad(criterion_embedding, ((0, 8 - C), (0, 0)))
    s1c = jnp.concatenate(
        [trans_s1[i] for i in range(C)], axis=1).astype(_BF16)
    s2c = jnp.pad(jnp.squeeze(trans_s2, -1), ((0, 8 - C), (0, 0)))
    wgb = w_gcn.astype(_BF16)

    layer1 = _make_tc_layer(final=False)
    layer2 = _make_tc_layer(final=True)

    y1 = _sc_spmm3(xall0, src_cat, dst_cat, val_cat)
    y1 = y1.reshape(2 * C, NPAD, HALF)
    x1, c1p, cmp_ = layer1(y1, wgb, W_gc_0.astype(_BF16), ce0, s1c, s2c,
                           W_rel_0, W_rel_1)

    y2 = _sc_spmm3(x1.reshape(2 * C * N, HALF), src_cat, dst_cat, val_cat)
    y2 = y2.reshape(2 * C, NPAD, HALF)
    (acc,) = layer2(y2, x1, p0[0], p0[1], p0[2],
                    wgb, W_gc_1.astype(_BF16), c1p, s1c, s2c)

    users = acc[:N_USERS]
    items = jnp.concatenate(
        [acc[N_USERS:], jnp.zeros((1, C, EMB), _F32)], axis=0)
    cris = tuple(cmp_[i:i + 1] for i in range(C))
    return (users, items) + cris


# speedup vs baseline: 1.0054x; 1.0054x over previous
import dataclasses

import jax
import jax.numpy as jnp
from jax import lax
from jax.experimental import pallas as pl
from jax.experimental.pallas import tpu as pltpu
from jax.experimental.pallas import tpu_sc as plsc

N_USERS = 25000
N_ITEMS = 25000
N = N_USERS + N_ITEMS
E = 800000
EMB = 64
HALF = 32
ATT = 32
C = 3

NC = 2
NS = 16
CHUNK = 128
SB = 40
NSB = 10
ROWS_PER_SUB = SB * NSB
EROWS = NS * ROWS_PER_SUB
EPAD = EROWS * CHUNK
ACC_PER_SUB = 3136
NPAD = NS * ACC_PER_SUB
G = 4
NGSB = SB // G
ZROWS = 32

_F32 = jnp.float32
_BF16 = jnp.bfloat16


def _leaky(x):
    return jnp.where(x >= 0, x, 0.3 * x)


def _sc_spmm_body(xall, src_hbm, dst_hbm, val_hbm, yall,
                  accum, sidx, didx, valv, rowsb, zbuf, gsem, ssem, isem):
    c = lax.axis_index("c")
    s = lax.axis_index("s")
    coff16 = jnp.full((16,), c * (C * N), jnp.int32)

    zero32 = jnp.zeros((32,), _BF16)

    @pl.loop(0, ZROWS)
    def _(r):
        zbuf[r, pl.ds(0, 32)] = zero32

    def zero_accum():
        @pl.loop(0, ACC_PER_SUB // ZROWS)
        def _(t):
            pltpu.sync_copy(
                zbuf, accum.at[pl.ds(s * ACC_PER_SUB + t * ZROWS, ZROWS)])

    zero_accum()
    plsc.subcore_barrier()

    gd = lax.GatherDimensionNumbers(
        offset_dims=(), collapsed_slice_dims=(0,), start_index_map=(0,))

    def lane_bcast(v16, k):
        return lax.gather(
            v16, jnp.full((16, 1), k, jnp.int32), gd, (1,),
            mode=lax.GatherScatterMode.PROMISE_IN_BOUNDS)

    @pl.loop(0, C)
    def _(i):
        base = i * EROWS + s * ROWS_PER_SUB

        def fire_gathers(g, slot):
            for j in range(G):
                pltpu.async_copy(
                    xall.at[sidx.at[g * G + j]],
                    rowsb.at[slot * G + j], gsem.at[slot])

        def drain(sem_slot):
            for j in range(G):
                pltpu.make_async_copy(
                    xall.at[pl.ds(0, CHUNK)], rowsb.at[j], sem_slot).wait()

        def scale(g, slot):
            for j in range(G):
                r = slot * G + j
                ir = g * G + j

                @pl.loop(0, CHUNK, step=16)
                def _(e0):
                    v16 = valv[ir, pl.ds(e0, 16)]
                    for k in range(16):
                        vb = lane_bcast(v16, k)
                        vb2 = plsc.pack(
                            vb, vb, format=plsc.PackFormat.INTERLEAVED)
                        e = e0 + k
                        rowsb[r, e, pl.ds(0, 32)] = (
                            rowsb[r, e, pl.ds(0, 32)] * vb2)

        def fire_scatters(g, slot):
            for j in range(G):
                pltpu.async_copy(
                    rowsb.at[slot * G + j], accum.at[didx.at[g * G + j]],
                    ssem.at[slot], add=True)

        def group_step(g, slot, first, last):
            other = 1 - slot
            drain(gsem.at[slot])
            scale(g, slot)
            if not first:
                drain(ssem.at[other])
            fire_scatters(g, slot)
            if not last:
                fire_gathers(g + 1, other)

        @pl.loop(0, NSB)
        def _(b):
            sb = base + b * SB
            d1 = pltpu.async_copy(src_hbm.at[pl.ds(sb, SB)], sidx, isem)
            d2 = pltpu.async_copy(dst_hbm.at[pl.ds(sb, SB)], didx, isem)
            d3 = pltpu.async_copy(val_hbm.at[pl.ds(sb, SB)], valv, isem)
            d1.wait()
            d2.wait()
            d3.wait()

            @pl.loop(0, SB)
            def _(r):
                @pl.loop(0, CHUNK, step=16)
                def _(k):
                    sidx[r, pl.ds(k, 16)] = sidx[r, pl.ds(k, 16)] + coff16

            fire_gathers(0, 0)
            group_step(0, 0, True, False)

            @pl.loop(1, NGSB - 1, step=2)
            def _(g):
                group_step(g, 1, False, False)
                group_step(g + 1, 0, False, False)

            group_step(NGSB - 1, 1, False, True)
            drain(ssem.at[1])

        plsc.subcore_barrier()
        off = (c * C + i) * NPAD + s * ACC_PER_SUB
        pltpu.sync_copy(
            accum.at[pl.ds(s * ACC_PER_SUB, ACC_PER_SUB)],
            yall.at[pl.ds(off, ACC_PER_SUB)])
        zero_accum()
        plsc.subcore_barrier()


def _sc_compiler_params():
    cp = pltpu.CompilerParams()
    fields = pltpu.CompilerParams.__dataclass_fields__
    if "needs_layout_passes" in fields:
        cp = dataclasses.replace(cp, needs_layout_passes=False)
    if "use_tc_tiling_on_sc" in fields:
        cp = dataclasses.replace(cp, use_tc_tiling_on_sc=False)
    return cp


def _sc_spmm3(xall, src_cat, dst_cat, val_cat):
    mesh = plsc.VectorSubcoreMesh(
        core_axis_name="c", subcore_axis_name="s",
        num_cores=NC, num_subcores=NS)
    fn = pl.kernel(
        _sc_spmm_body,
        out_type=jax.ShapeDtypeStruct((NC * C * NPAD, HALF), _BF16),
        mesh=mesh,
        scratch_types=[
            pltpu.VMEM_SHARED((NPAD, HALF), _BF16),
            pltpu.VMEM((SB, CHUNK), jnp.int32),
            pltpu.VMEM((SB, CHUNK), jnp.int32),
            pltpu.VMEM((SB, CHUNK), _F32),
            pltpu.VMEM((2 * G, CHUNK, HALF), _BF16),
            pltpu.VMEM((ZROWS, HALF), _BF16),
            pltpu.SemaphoreType.DMA((2,)),
            pltpu.SemaphoreType.DMA((2,)),
            pltpu.SemaphoreType.DMA,
        ],
        name="sc_spmm3",
        compiler_params=_sc_compiler_params(),
    )
    return fn(xall, src_cat, dst_cat, val_cat)


_BLK = 2000
_GRID = N // _BLK


def _attention_mix(g, s1v, s2v):
    t = [jnp.tanh(jnp.dot(g[cc].astype(_BF16), s1v,
                          preferred_element_type=_F32))
         for cc in range(C)]
    outs = []
    for i in range(C):
        s2row = s2v[i][None, :]
        l = [jnp.sum(t[cc][:, 32 * i:32 * i + 32] * s2row,
                     axis=1, keepdims=True) for cc in range(C)]
        m = jnp.maximum(jnp.maximum(l[0], l[1]), l[2])
        w = [jnp.exp(x - m) for x in l]
        tot = w[0] + w[1] + w[2]
        outs.append(
            _leaky((w[0] * g[0] + w[1] * g[1] + w[2] * g[2]) / tot))
    return outs


def _gcn_transform(y3, wgv, wgcv, cev):
    g = []
    for cc in range(C):
        e = jnp.concatenate([y3[cc], y3[C + cc]], axis=1)
        h = jnp.dot(e, wgv, preferred_element_type=_F32)
        h = (h * cev[cc][None, :]).astype(_BF16)
        h = _leaky(jnp.dot(h, wgcv, preferred_element_type=_F32))
        g.append(h)
    return g


def _make_tc_layer(final):
    def body(*refs):
        if final:
            y3, x1, p00, p01, p02, wg, wgc, ce, s1c, s2c = refs[:10]
            outs = refs[10:]
            p0 = (p00, p01, p02)
        else:
            y3, wg, wgc, ce, s1c, s2c, wr0, wr1 = refs[:8]
            outs = refs[8:]

        g = _gcn_transform(y3, wg[...], wgc[...], ce[...])
        mix = _attention_mix(g, s1c[...], s2c[...])

        for i in range(C):
            if final:
                p1 = jnp.concatenate(
                    [x1[i], x1[C + i]], axis=1).astype(_F32)
                outs[i][...] = (p0[i][...] + p1 + mix[i]) * (1.0 / 3.0)
            else:
                ob = mix[i].astype(_BF16)
                outs[0][i] = ob[:, :HALF]
                outs[0][C + i] = ob[:, HALF:]

        if not final:
            @pl.when(pl.program_id(0) == 0)
            def _():
                c0 = ce[...].astype(_F32)
                c1 = _leaky(jnp.dot(c0, wr0[...],
                                    preferred_element_type=_F32))
                c2 = _leaky(jnp.dot(c1, wr1[...],
                                    preferred_element_type=_F32))
                outs[1][...] = c1
                outs[2][...] = (c0 + c1 + c2) * (1.0 / 3.0)

    row_spec = lambda w: pl.BlockSpec((_BLK, w), lambda b: (b, 0))
    wt_spec = lambda a, b_: pl.BlockSpec((a, b_), lambda b: (0, 0))
    y3_spec = pl.BlockSpec((2 * C, _BLK, HALF), lambda b: (0, b, 0))

    wt_specs = [wt_spec(EMB, EMB), wt_spec(EMB, EMB),
                wt_spec(8, EMB), wt_spec(EMB, 96), wt_spec(8, 32)]
    if final:
        in_specs = [y3_spec, y3_spec] + [row_spec(EMB)] * 3 + wt_specs
        out_specs = [row_spec(EMB)] * 3
        out_shape = [jax.ShapeDtypeStruct((N, EMB), _F32)] * 3
    else:
        in_specs = [y3_spec] + wt_specs + [wt_spec(EMB, EMB)] * 2
        out_specs = [
            pl.BlockSpec((2 * C, _BLK, HALF), lambda b: (0, b, 0)),
            wt_spec(8, EMB), wt_spec(8, EMB)]
        out_shape = [
            jax.ShapeDtypeStruct((2 * C, N, HALF), _BF16),
            jax.ShapeDtypeStruct((8, EMB), _F32),
            jax.ShapeDtypeStruct((8, EMB), _F32)]

    return pl.pallas_call(
        body,
        grid=(_GRID,),
        in_specs=in_specs,
        out_specs=out_specs,
        out_shape=out_shape,
    )


def kernel(adj_idx_0, adj_val_0, adj_idx_1, adj_val_1, adj_idx_2, adj_val_2,
           user_embedding, item_embedding, criterion_embedding, w_gcn,
           W_gc_0, W_gc_1, W_rel_0, W_rel_1, trans_s1, trans_s2):
    pad = EPAD - E
    adj = ((adj_idx_0, adj_val_0), (adj_idx_1, adj_val_1),
           (adj_idx_2, adj_val_2))
    src_cat = jnp.concatenate(
        [(jnp.pad(ai[1], (0, pad)) + i * N).reshape(EROWS, CHUNK)
         for i, (ai, _) in enumerate(adj)], axis=0)
    dst_cat = jnp.concatenate(
        [jnp.pad(ai[0], (0, pad)).reshape(EROWS, CHUNK)
         for ai, _ in adj], axis=0)
    val_cat = jnp.concatenate(
        [jnp.pad(av, (0, pad)).reshape(EROWS, CHUNK)
         for _, av in adj], axis=0)

    pre = jnp.concatenate([user_embedding, item_embedding], axis=0)
    p0 = [pre[:, i, :] for i in range(C)]
    xall0 = jnp.concatenate(
        [p0[i][:, :HALF] for i in range(C)]
        + [p0[i][:, HALF:] for i in range(C)], axis=0).astype(_BF16)

    ce0 = jnp.pad(criterion_embedding, ((0, 8 - C), (0, 0)))
    s1c = jnp.concatenate(
        [trans_s1[i] for i in range(C)], axis=1).astype(_BF16)
    s2c = jnp.pad(jnp.squeeze(trans_s2, -1), ((0, 8 - C), (0, 0)))
    wgb = w_gcn.astype(_BF16)

    layer1 = _make_tc_layer(final=False)
    layer2 = _make_tc_layer(final=True)

    y1 = _sc_spmm3(xall0, src_cat, dst_cat, val_cat)
    y1 = y1.reshape(2 * C, NPAD, HALF)
    x1, c1p, cmp_ = layer1(y1, wgb, W_gc_0.astype(_BF16), ce0, s1c, s2c,
                           W_rel_0, W_rel_1)

    y2 = _sc_spmm3(x1.reshape(2 * C * N, HALF), src_cat, dst_cat, val_cat)
    y2 = y2.reshape(2 * C, NPAD, HALF)
    accs = layer2(y2, x1, p0[0], p0[1], p0[2],
                  wgb, W_gc_1.astype(_BF16), c1p, s1c, s2c)

    acc = jnp.stack(accs, axis=1)
    users = acc[:N_USERS]
    items = jnp.concatenate(
        [acc[N_USERS:], jnp.zeros((1, C, EMB), _F32)], axis=0)
    cris = tuple(cmp_[i:i + 1] for i in range(C))
    return (users, items) + cris


# speedup vs baseline: 1.0755x; 1.0697x over previous
import dataclasses

import jax
import jax.numpy as jnp
from jax import lax
from jax.experimental import pallas as pl
from jax.experimental.pallas import tpu as pltpu
from jax.experimental.pallas import tpu_sc as plsc

N_USERS = 25000
N_ITEMS = 25000
N = N_USERS + N_ITEMS
E = 800000
EMB = 64
HALF = 32
ATT = 32
C = 3

NC = 2
NS = 16
CHUNK = 128
SB = 40
NSB = 10
ROWS_PER_SUB = SB * NSB
EROWS = NS * ROWS_PER_SUB
EPAD = EROWS * CHUNK
ACC_PER_SUB = 3136
NPAD = NS * ACC_PER_SUB
G = 4
NGSB = SB // G
ZROWS = 32

_F32 = jnp.float32
_BF16 = jnp.bfloat16


def _leaky(x):
    return jnp.where(x >= 0, x, 0.3 * x)


def _sc_spmm_body(xall, src_hbm, dst_hbm, val_hbm, yall,
                  accum, sidx, didx, valv, rowsb, zbuf, gsem, ssem, isem):
    c = lax.axis_index("c")
    s = lax.axis_index("s")
    coff16 = jnp.full((16,), c * (C * N), jnp.int32)

    zero32 = jnp.zeros((32,), _BF16)

    @pl.loop(0, ZROWS)
    def _(r):
        zbuf[r, pl.ds(0, 32)] = zero32

    def zero_accum():
        @pl.loop(0, ACC_PER_SUB // ZROWS)
        def _(t):
            pltpu.sync_copy(
                zbuf, accum.at[pl.ds(s * ACC_PER_SUB + t * ZROWS, ZROWS)])

    zero_accum()
    plsc.subcore_barrier()

    gd = lax.GatherDimensionNumbers(
        offset_dims=(), collapsed_slice_dims=(0,), start_index_map=(0,))

    def lane_bcast(v16, k):
        return lax.gather(
            v16, jnp.full((16, 1), k, jnp.int32), gd, (1,),
            mode=lax.GatherScatterMode.PROMISE_IN_BOUNDS)

    @pl.loop(0, C)
    def _(i):
        base = i * EROWS + s * ROWS_PER_SUB

        def fire_gathers(g, slot):
            for j in range(G):
                pltpu.async_copy(
                    xall.at[sidx.at[g * G + j]],
                    rowsb.at[slot * G + j], gsem.at[slot])

        def drain(sem_slot):
            for j in range(G):
                pltpu.make_async_copy(
                    xall.at[pl.ds(0, CHUNK)], rowsb.at[j], sem_slot).wait()

        def scale(g, slot):
            for j in range(G):
                r = slot * G + j
                ir = g * G + j

                @pl.loop(0, CHUNK, step=16)
                def _(e0):
                    v16 = valv[ir, pl.ds(e0, 16)]
                    for k in range(16):
                        vb = lane_bcast(v16, k)
                        vb2 = plsc.pack(
                            vb, vb, format=plsc.PackFormat.INTERLEAVED)
                        e = e0 + k
                        rowsb[r, e, pl.ds(0, 32)] = (
                            rowsb[r, e, pl.ds(0, 32)] * vb2)

        def fire_scatters(g, slot):
            for j in range(G):
                pltpu.async_copy(
                    rowsb.at[slot * G + j], accum.at[didx.at[g * G + j]],
                    ssem.at[slot], add=True)

        def group_step(g, slot, first, last):
            other = 1 - slot
            drain(gsem.at[slot])
            scale(g, slot)
            if not first:
                drain(ssem.at[other])
            fire_scatters(g, slot)
            if not last:
                fire_gathers(g + 1, other)

        @pl.loop(0, NSB)
        def _(b):
            sb = base + b * SB
            d1 = pltpu.async_copy(src_hbm.at[pl.ds(sb, SB)], sidx, isem)
            d2 = pltpu.async_copy(dst_hbm.at[pl.ds(sb, SB)], didx, isem)
            d3 = pltpu.async_copy(val_hbm.at[pl.ds(sb, SB)], valv, isem)
            d1.wait()
            d2.wait()
            d3.wait()

            @pl.loop(0, SB)
            def _(r):
                @pl.loop(0, CHUNK, step=16)
                def _(k):
                    sidx[r, pl.ds(k, 16)] = sidx[r, pl.ds(k, 16)] + coff16

            fire_gathers(0, 0)
            group_step(0, 0, True, False)

            @pl.loop(1, NGSB - 1, step=2)
            def _(g):
                group_step(g, 1, False, False)
                group_step(g + 1, 0, False, False)

            group_step(NGSB - 1, 1, False, True)
            drain(ssem.at[1])

        plsc.subcore_barrier()
        off = (c * C + i) * NPAD + s * ACC_PER_SUB
        pltpu.sync_copy(
            accum.at[pl.ds(s * ACC_PER_SUB, ACC_PER_SUB)],
            yall.at[pl.ds(off, ACC_PER_SUB)])
        zero_accum()
        plsc.subcore_barrier()


def _sc_compiler_params():
    cp = pltpu.CompilerParams()
    fields = pltpu.CompilerParams.__dataclass_fields__
    if "needs_layout_passes" in fields:
        cp = dataclasses.replace(cp, needs_layout_passes=False)
    if "use_tc_tiling_on_sc" in fields:
        cp = dataclasses.replace(cp, use_tc_tiling_on_sc=False)
    return cp


def _sc_spmm3(xall, src_cat, dst_cat, val_cat):
    mesh = plsc.VectorSubcoreMesh(
        core_axis_name="c", subcore_axis_name="s",
        num_cores=NC, num_subcores=NS)
    fn = pl.kernel(
        _sc_spmm_body,
        out_type=jax.ShapeDtypeStruct((NC * C * NPAD, HALF), _BF16),
        mesh=mesh,
        scratch_types=[
            pltpu.VMEM_SHARED((NPAD, HALF), _BF16),
            pltpu.VMEM((SB, CHUNK), jnp.int32),
            pltpu.VMEM((SB, CHUNK), jnp.int32),
            pltpu.VMEM((SB, CHUNK), _F32),
            pltpu.VMEM((2 * G, CHUNK, HALF), _BF16),
            pltpu.VMEM((ZROWS, HALF), _BF16),
            pltpu.SemaphoreType.DMA((2,)),
            pltpu.SemaphoreType.DMA((2,)),
            pltpu.SemaphoreType.DMA,
        ],
        name="sc_spmm3",
        compiler_params=_sc_compiler_params(),
    )
    return fn(xall, src_cat, dst_cat, val_cat)


_BLK = 2000
_GRID = N // _BLK


def _attention_mix(g, s1v, s2v):
    t = [jnp.tanh(jnp.dot(g[cc], s1v, preferred_element_type=_F32))
         for cc in range(C)]
    outs = []
    for i in range(C):
        s2row = s2v[i][None, :]
        l = [jnp.sum(t[cc][:, 32 * i:32 * i + 32] * s2row,
                     axis=1, keepdims=True) for cc in range(C)]
        m = jnp.maximum(jnp.maximum(l[0], l[1]), l[2])
        w = [jnp.exp(x - m) for x in l]
        tot = w[0] + w[1] + w[2]
        outs.append(
            _leaky((w[0] * g[0] + w[1] * g[1] + w[2] * g[2]) / tot))
    return outs


def _gcn_transform(y3, wgv, wgcv, cev):
    g = []
    for cc in range(C):
        e = jnp.concatenate([y3[cc], y3[C + cc]], axis=1).astype(_F32)
        h = jnp.dot(e, wgv, preferred_element_type=_F32)
        h = h * cev[cc][None, :]
        h = _leaky(jnp.dot(h, wgcv, preferred_element_type=_F32))
        g.append(h)
    return g


def _make_tc_layer(final):
    def body(*refs):
        if final:
            y3, x1, p00, p01, p02, wg, wgc, ce, s1c, s2c = refs[:10]
            outs = refs[10:]
            p0 = (p00, p01, p02)
        else:
            y3, wg, wgc, ce, s1c, s2c, wr0, wr1 = refs[:8]
            outs = refs[8:]

        g = _gcn_transform(y3, wg[...], wgc[...], ce[...])
        mix = _attention_mix(g, s1c[...], s2c[...])

        for i in range(C):
            if final:
                p1 = jnp.concatenate(
                    [x1[i], x1[C + i]], axis=1).astype(_F32)
                outs[i][...] = (p0[i][...] + p1 + mix[i]) * (1.0 / 3.0)
            else:
                ob = mix[i].astype(_BF16)
                outs[0][i] = ob[:, :HALF]
                outs[0][C + i] = ob[:, HALF:]

        if not final:
            @pl.when(pl.program_id(0) == 0)
            def _():
                c0 = ce[...].astype(_F32)
                c1 = _leaky(jnp.dot(c0, wr0[...],
                                    preferred_element_type=_F32))
                c2 = _leaky(jnp.dot(c1, wr1[...],
                                    preferred_element_type=_F32))
                outs[1][...] = c1
                outs[2][...] = (c0 + c1 + c2) * (1.0 / 3.0)

    row_spec = lambda w: pl.BlockSpec((_BLK, w), lambda b: (b, 0))
    wt_spec = lambda a, b_: pl.BlockSpec((a, b_), lambda b: (0, 0))
    y3_spec = pl.BlockSpec((2 * C, _BLK, HALF), lambda b: (0, b, 0))

    wt_specs = [wt_spec(EMB, EMB), wt_spec(EMB, EMB),
                wt_spec(8, EMB), wt_spec(EMB, 96), wt_spec(8, 32)]
    if final:
        in_specs = [y3_spec, y3_spec] + [row_spec(EMB)] * 3 + wt_specs
        out_specs = [row_spec(EMB)] * 3
        out_shape = [jax.ShapeDtypeStruct((N, EMB), _F32)] * 3
    else:
        in_specs = [y3_spec] + wt_specs + [wt_spec(EMB, EMB)] * 2
        out_specs = [
            pl.BlockSpec((2 * C, _BLK, HALF), lambda b: (0, b, 0)),
            wt_spec(8, EMB), wt_spec(8, EMB)]
        out_shape = [
            jax.ShapeDtypeStruct((2 * C, N, HALF), _BF16),
            jax.ShapeDtypeStruct((8, EMB), _F32),
            jax.ShapeDtypeStruct((8, EMB), _F32)]

    return pl.pallas_call(
        body,
        grid=(_GRID,),
        in_specs=in_specs,
        out_specs=out_specs,
        out_shape=out_shape,
    )


def kernel(adj_idx_0, adj_val_0, adj_idx_1, adj_val_1, adj_idx_2, adj_val_2,
           user_embedding, item_embedding, criterion_embedding, w_gcn,
           W_gc_0, W_gc_1, W_rel_0, W_rel_1, trans_s1, trans_s2):
    pad = EPAD - E
    adj = ((adj_idx_0, adj_val_0), (adj_idx_1, adj_val_1),
           (adj_idx_2, adj_val_2))
    src_cat = jnp.concatenate(
        [(jnp.pad(ai[1], (0, pad)) + i * N).reshape(EROWS, CHUNK)
         for i, (ai, _) in enumerate(adj)], axis=0)
    dst_cat = jnp.concatenate(
        [jnp.pad(ai[0], (0, pad)).reshape(EROWS, CHUNK)
         for ai, _ in adj], axis=0)
    val_cat = jnp.concatenate(
        [jnp.pad(av, (0, pad)).reshape(EROWS, CHUNK)
         for _, av in adj], axis=0)

    pre = jnp.concatenate([user_embedding, item_embedding], axis=0)
    p0 = [pre[:, i, :] for i in range(C)]
    xall0 = jnp.concatenate(
        [p0[i][:, :HALF] for i in range(C)]
        + [p0[i][:, HALF:] for i in range(C)], axis=0).astype(_BF16)

    ce0 = jnp.pad(criterion_embedding, ((0, 8 - C), (0, 0)))
    s1c = jnp.concatenate(
        [trans_s1[i] for i in range(C)], axis=1)
    s2c = jnp.pad(jnp.squeeze(trans_s2, -1), ((0, 8 - C), (0, 0)))

    layer1 = _make_tc_layer(final=False)
    layer2 = _make_tc_layer(final=True)

    y1 = _sc_spmm3(xall0, src_cat, dst_cat, val_cat)
    y1 = y1.reshape(2 * C, NPAD, HALF)
    x1, c1p, cmp_ = layer1(y1, w_gcn, W_gc_0, ce0, s1c, s2c,
                           W_rel_0, W_rel_1)

    y2 = _sc_spmm3(x1.reshape(2 * C * N, HALF), src_cat, dst_cat, val_cat)
    y2 = y2.reshape(2 * C, NPAD, HALF)
    accs = layer2(y2, x1, p0[0], p0[1], p0[2],
                  w_gcn, W_gc_1, c1p, s1c, s2c)

    acc = jnp.stack(accs, axis=1)
    users = acc[:N_USERS]
    items = jnp.concatenate(
        [acc[N_USERS:], jnp.zeros((1, C, EMB), _F32)], axis=0)
    cris = tuple(cmp_[i:i + 1] for i in range(C))
    return (users, items) + cris


# speedup vs baseline: 1.0839x; 1.0078x over previous
import dataclasses

import jax
import jax.numpy as jnp
from jax import lax
from jax.experimental import pallas as pl
from jax.experimental.pallas import tpu as pltpu
from jax.experimental.pallas import tpu_sc as plsc

N_USERS = 25000
N_ITEMS = 25000
N = N_USERS + N_ITEMS
E = 800000
EMB = 64
HALF = 32
ATT = 32
C = 3

NC = 2
NS = 16
CHUNK = 128
SB = 40
NSB = 10
ROWS_PER_SUB = SB * NSB
EROWS = NS * ROWS_PER_SUB
EPAD = EROWS * CHUNK
ACC_PER_SUB = 3136
NPAD = NS * ACC_PER_SUB
G = 4
NGSB = SB // G
ZROWS = 32

_F32 = jnp.float32
_BF16 = jnp.bfloat16


def _leaky(x):
    return jnp.where(x >= 0, x, 0.3 * x)


def _sc_spmm_body(xall, src_hbm, dst_hbm, val_hbm, yall,
                  accum, sidx, didx, valv, rowsb, zbuf, gsem, ssem, isem):
    c = lax.axis_index("c")
    s = lax.axis_index("s")

    zero32 = jnp.zeros((32,), _BF16)

    @pl.loop(0, ZROWS)
    def _(r):
        zbuf[r, pl.ds(0, 32)] = zero32

    def zero_accum():
        @pl.loop(0, ACC_PER_SUB // ZROWS)
        def _(t):
            pltpu.sync_copy(
                zbuf, accum.at[pl.ds(s * ACC_PER_SUB + t * ZROWS, ZROWS)])

    zero_accum()
    plsc.subcore_barrier()

    gd = lax.GatherDimensionNumbers(
        offset_dims=(), collapsed_slice_dims=(0,), start_index_map=(0,))

    def lane_bcast(v16, k):
        return lax.gather(
            v16, jnp.full((16, 1), k, jnp.int32), gd, (1,),
            mode=lax.GatherScatterMode.PROMISE_IN_BOUNDS)

    @pl.loop(0, C)
    def _(i):
        base = i * EROWS + s * ROWS_PER_SUB
        plane = c * C + i

        def fire_gathers(g, slot):
            for j in range(G):
                pltpu.async_copy(
                    xall.at[plane].at[sidx.at[g * G + j]],
                    rowsb.at[slot * G + j], gsem.at[slot])

        def drain(sem_slot):
            for j in range(G):
                pltpu.make_async_copy(
                    xall.at[0].at[pl.ds(0, CHUNK)], rowsb.at[j],
                    sem_slot).wait()

        def scale(g, slot):
            for j in range(G):
                r = slot * G + j
                ir = g * G + j

                @pl.loop(0, CHUNK, step=16)
                def _(e0):
                    v16 = valv[ir, pl.ds(e0, 16)]
                    for k in range(16):
                        vb = lane_bcast(v16, k)
                        vb2 = plsc.pack(
                            vb, vb, format=plsc.PackFormat.INTERLEAVED)
                        e = e0 + k
                        rowsb[r, e, pl.ds(0, 32)] = (
                            rowsb[r, e, pl.ds(0, 32)] * vb2)

        def fire_scatters(g, slot):
            for j in range(G):
                pltpu.async_copy(
                    rowsb.at[slot * G + j], accum.at[didx.at[g * G + j]],
                    ssem.at[slot], add=True)

        def group_step(g, slot, first, last):
            other = 1 - slot
            drain(gsem.at[slot])
            scale(g, slot)
            if not first:
                drain(ssem.at[other])
            fire_scatters(g, slot)
            if not last:
                fire_gathers(g + 1, other)

        @pl.loop(0, NSB)
        def _(b):
            sb = base + b * SB
            d1 = pltpu.async_copy(src_hbm.at[pl.ds(sb, SB)], sidx, isem)
            d2 = pltpu.async_copy(dst_hbm.at[pl.ds(sb, SB)], didx, isem)
            d3 = pltpu.async_copy(val_hbm.at[pl.ds(sb, SB)], valv, isem)
            d1.wait()
            d2.wait()
            d3.wait()

            fire_gathers(0, 0)
            group_step(0, 0, True, False)

            @pl.loop(1, NGSB - 1, step=2)
            def _(g):
                group_step(g, 1, False, False)
                group_step(g + 1, 0, False, False)

            group_step(NGSB - 1, 1, False, True)
            drain(ssem.at[1])

        plsc.subcore_barrier()
        pltpu.sync_copy(
            accum.at[pl.ds(s * ACC_PER_SUB, ACC_PER_SUB)],
            yall.at[plane].at[pl.ds(s * ACC_PER_SUB, ACC_PER_SUB)])
        zero_accum()
        plsc.subcore_barrier()


def _sc_compiler_params():
    cp = pltpu.CompilerParams()
    fields = pltpu.CompilerParams.__dataclass_fields__
    if "needs_layout_passes" in fields:
        cp = dataclasses.replace(cp, needs_layout_passes=False)
    if "use_tc_tiling_on_sc" in fields:
        cp = dataclasses.replace(cp, use_tc_tiling_on_sc=False)
    return cp


def _sc_spmm3(xall, src_cat, dst_cat, val_cat):
    mesh = plsc.VectorSubcoreMesh(
        core_axis_name="c", subcore_axis_name="s",
        num_cores=NC, num_subcores=NS)
    fn = pl.kernel(
        _sc_spmm_body,
        out_type=jax.ShapeDtypeStruct((NC * C, NPAD, HALF), _BF16),
        mesh=mesh,
        scratch_types=[
            pltpu.VMEM_SHARED((NPAD, HALF), _BF16),
            pltpu.VMEM((SB, CHUNK), jnp.int32),
            pltpu.VMEM((SB, CHUNK), jnp.int32),
            pltpu.VMEM((SB, CHUNK), _F32),
            pltpu.VMEM((2 * G, CHUNK, HALF), _BF16),
            pltpu.VMEM((ZROWS, HALF), _BF16),
            pltpu.SemaphoreType.DMA((2,)),
            pltpu.SemaphoreType.DMA((2,)),
            pltpu.SemaphoreType.DMA,
        ],
        name="sc_spmm3",
        compiler_params=_sc_compiler_params(),
    )
    return fn(xall, src_cat, dst_cat, val_cat)


_BLK = 2000
_GRID = N // _BLK


def _attention_mix(g, s1v, s2v):
    t = [jnp.tanh(jnp.dot(g[cc], s1v, preferred_element_type=_F32))
         for cc in range(C)]
    outs = []
    for i in range(C):
        s2row = s2v[i][None, :]
        l = [jnp.sum(t[cc][:, 32 * i:32 * i + 32] * s2row,
                     axis=1, keepdims=True) for cc in range(C)]
        m = jnp.maximum(jnp.maximum(l[0], l[1]), l[2])
        w = [jnp.exp(x - m) for x in l]
        tot = w[0] + w[1] + w[2]
        outs.append(
            _leaky((w[0] * g[0] + w[1] * g[1] + w[2] * g[2]) / tot))
    return outs


def _gcn_transform(y3, wgv, wgcv, cev):
    g = []
    for cc in range(C):
        e = jnp.concatenate([y3[cc], y3[C + cc]], axis=1).astype(_F32)
        h = jnp.dot(e, wgv, preferred_element_type=_F32)
        h = h * cev[cc][None, :]
        h = _leaky(jnp.dot(h, wgcv, preferred_element_type=_F32))
        g.append(h)
    return g


def _make_tc_layer(final):
    def body(*refs):
        if final:
            y3, x1, p00, p01, p02, wg, wgc, ce, s1c, s2c = refs[:10]
            outs = refs[10:]
            p0 = (p00, p01, p02)
        else:
            y3, wg, wgc, ce, s1c, s2c, wr0, wr1 = refs[:8]
            outs = refs[8:]

        g = _gcn_transform(y3, wg[...], wgc[...], ce[...])
        mix = _attention_mix(g, s1c[...], s2c[...])

        for i in range(C):
            if final:
                p1 = jnp.concatenate(
                    [x1[i], x1[C + i]], axis=1).astype(_F32)
                outs[i][...] = (p0[i][...] + p1 + mix[i]) * (1.0 / 3.0)
            else:
                ob = mix[i].astype(_BF16)
                outs[0][i] = ob[:, :HALF]
                outs[0][C + i] = ob[:, HALF:]

        if not final:
            @pl.when(pl.program_id(0) == 0)
            def _():
                c0 = ce[...].astype(_F32)
                c1 = _leaky(jnp.dot(c0, wr0[...],
                                    preferred_element_type=_F32))
                c2 = _leaky(jnp.dot(c1, wr1[...],
                                    preferred_element_type=_F32))
                outs[1][...] = c1
                outs[2][...] = (c0 + c1 + c2) * (1.0 / 3.0)

    row_spec = lambda w: pl.BlockSpec((_BLK, w), lambda b: (b, 0))
    wt_spec = lambda a, b_: pl.BlockSpec((a, b_), lambda b: (0, 0))
    y3_spec = pl.BlockSpec((2 * C, _BLK, HALF), lambda b: (0, b, 0))

    wt_specs = [wt_spec(EMB, EMB), wt_spec(EMB, EMB),
                wt_spec(8, EMB), wt_spec(EMB, 96), wt_spec(8, 32)]
    if final:
        in_specs = [y3_spec, y3_spec] + [row_spec(EMB)] * 3 + wt_specs
        out_specs = [row_spec(EMB)] * 3
        out_shape = [jax.ShapeDtypeStruct((N, EMB), _F32)] * 3
    else:
        in_specs = [y3_spec] + wt_specs + [wt_spec(EMB, EMB)] * 2
        out_specs = [
            pl.BlockSpec((2 * C, _BLK, HALF), lambda b: (0, b, 0)),
            wt_spec(8, EMB), wt_spec(8, EMB)]
        out_shape = [
            jax.ShapeDtypeStruct((2 * C, N, HALF), _BF16),
            jax.ShapeDtypeStruct((8, EMB), _F32),
            jax.ShapeDtypeStruct((8, EMB), _F32)]

    return pl.pallas_call(
        body,
        grid=(_GRID,),
        in_specs=in_specs,
        out_specs=out_specs,
        out_shape=out_shape,
    )


def kernel(adj_idx_0, adj_val_0, adj_idx_1, adj_val_1, adj_idx_2, adj_val_2,
           user_embedding, item_embedding, criterion_embedding, w_gcn,
           W_gc_0, W_gc_1, W_rel_0, W_rel_1, trans_s1, trans_s2):
    pad = EPAD - E
    adj = ((adj_idx_0, adj_val_0), (adj_idx_1, adj_val_1),
           (adj_idx_2, adj_val_2))
    src_cat = jnp.concatenate(
        [jnp.pad(ai[1], (0, pad)).reshape(EROWS, CHUNK)
         for ai, _ in adj], axis=0)
    dst_cat = jnp.concatenate(
        [jnp.pad(ai[0], (0, pad)).reshape(EROWS, CHUNK)
         for ai, _ in adj], axis=0)
    val_cat = jnp.concatenate(
        [jnp.pad(av, (0, pad)).reshape(EROWS, CHUNK)
         for _, av in adj], axis=0)

    pre = jnp.concatenate([user_embedding, item_embedding], axis=0)
    p0 = [pre[:, i, :] for i in range(C)]
    xall0 = jnp.stack(
        [p0[i][:, :HALF] for i in range(C)]
        + [p0[i][:, HALF:] for i in range(C)], axis=0).astype(_BF16)

    ce0 = jnp.pad(criterion_embedding, ((0, 8 - C), (0, 0)))
    s1c = jnp.concatenate(
        [trans_s1[i] for i in range(C)], axis=1)
    s2c = jnp.pad(jnp.squeeze(trans_s2, -1), ((0, 8 - C), (0, 0)))

    layer1 = _make_tc_layer(final=False)
    layer2 = _make_tc_layer(final=True)

    y1 = _sc_spmm3(xall0, src_cat, dst_cat, val_cat)
    x1, c1p, cmp_ = layer1(y1, w_gcn, W_gc_0, ce0, s1c, s2c,
                           W_rel_0, W_rel_1)

    y2 = _sc_spmm3(x1, src_cat, dst_cat, val_cat)
    accs = layer2(y2, x1, p0[0], p0[1], p0[2],
                  w_gcn, W_gc_1, c1p, s1c, s2c)

    acc = jnp.stack(accs, axis=1)
    users = acc[:N_USERS]
    items = jnp.concatenate(
        [acc[N_USERS:], jnp.zeros((1, C, EMB), _F32)], axis=0)
    cris = tuple(cmp_[i:i + 1] for i in range(C))
    return (users, items) + cris


# speedup vs baseline: 1.1910x; 1.0988x over previous
import dataclasses

import jax
import jax.numpy as jnp
from jax import lax
from jax.experimental import pallas as pl
from jax.experimental.pallas import tpu as pltpu
from jax.experimental.pallas import tpu_sc as plsc

N_USERS = 25000
N_ITEMS = 25000
N = N_USERS + N_ITEMS
E = 800000
EMB = 64
HALF = 32
ATT = 32
C = 3

NC = 2
NS = 16
CHUNK = 128
SB = 40
NSB = 10
ROWS_PER_SUB = SB * NSB
EROWS = NS * ROWS_PER_SUB
EPAD = EROWS * CHUNK
ACC_PER_SUB = 3136
NPAD = NS * ACC_PER_SUB
G = 10
NGSB = SB // G
ZROWS = 32

_F32 = jnp.float32
_BF16 = jnp.bfloat16


def _leaky(x):
    return jnp.where(x >= 0, x, 0.3 * x)


def _sc_spmm_body(xall, src_hbm, dst_hbm, val_hbm, yall,
                  accum, sidx, didx, valv, rowsb, zbuf, gsem, ssem, isem):
    c = lax.axis_index("c")
    s = lax.axis_index("s")

    zero32 = jnp.zeros((32,), _BF16)

    @pl.loop(0, ZROWS)
    def _(r):
        zbuf[r, pl.ds(0, 32)] = zero32

    def zero_accum():
        @pl.loop(0, ACC_PER_SUB // ZROWS)
        def _(t):
            pltpu.sync_copy(
                zbuf, accum.at[pl.ds(s * ACC_PER_SUB + t * ZROWS, ZROWS)])

    zero_accum()
    plsc.subcore_barrier()

    gd = lax.GatherDimensionNumbers(
        offset_dims=(), collapsed_slice_dims=(0,), start_index_map=(0,))

    def lane_bcast(v16, k):
        return lax.gather(
            v16, jnp.full((16, 1), k, jnp.int32), gd, (1,),
            mode=lax.GatherScatterMode.PROMISE_IN_BOUNDS)

    @pl.loop(0, C)
    def _(i):
        base = i * EROWS + s * ROWS_PER_SUB
        plane = c * C + i

        def fire_gathers(g, slot):
            for j in range(G):
                pltpu.async_copy(
                    xall.at[plane].at[sidx.at[g * G + j]],
                    rowsb.at[slot * G + j], gsem.at[slot])

        def drain(sem_slot):
            for j in range(G):
                pltpu.make_async_copy(
                    xall.at[0].at[pl.ds(0, CHUNK)], rowsb.at[j],
                    sem_slot).wait()

        def scale(g, slot):
            for j in range(G):
                r = slot * G + j
                ir = g * G + j

                @pl.loop(0, CHUNK, step=16)
                def _(e0):
                    v16 = valv[ir, pl.ds(e0, 16)]
                    for k in range(16):
                        vb = lane_bcast(v16, k)
                        vb2 = plsc.pack(
                            vb, vb, format=plsc.PackFormat.INTERLEAVED)
                        e = e0 + k
                        rowsb[r, e, pl.ds(0, 32)] = (
                            rowsb[r, e, pl.ds(0, 32)] * vb2)

        def fire_scatters(g, slot):
            for j in range(G):
                pltpu.async_copy(
                    rowsb.at[slot * G + j], accum.at[didx.at[g * G + j]],
                    ssem.at[slot], add=True)

        def group_step(g, slot, first, last):
            other = 1 - slot
            drain(gsem.at[slot])
            scale(g, slot)
            if not first:
                drain(ssem.at[other])
            fire_scatters(g, slot)
            if not last:
                fire_gathers(g + 1, other)

        @pl.loop(0, NSB)
        def _(b):
            sb = base + b * SB
            d1 = pltpu.async_copy(src_hbm.at[pl.ds(sb, SB)], sidx, isem)
            d2 = pltpu.async_copy(dst_hbm.at[pl.ds(sb, SB)], didx, isem)
            d3 = pltpu.async_copy(val_hbm.at[pl.ds(sb, SB)], valv, isem)
            d1.wait()
            d2.wait()
            d3.wait()

            fire_gathers(0, 0)
            group_step(0, 0, True, False)

            @pl.loop(1, NGSB - 1, step=2)
            def _(g):
                group_step(g, 1, False, False)
                group_step(g + 1, 0, False, False)

            group_step(NGSB - 1, 1, False, True)
            drain(ssem.at[1])

        plsc.subcore_barrier()
        pltpu.sync_copy(
            accum.at[pl.ds(s * ACC_PER_SUB, ACC_PER_SUB)],
            yall.at[plane].at[pl.ds(s * ACC_PER_SUB, ACC_PER_SUB)])
        zero_accum()
        plsc.subcore_barrier()


def _sc_compiler_params():
    cp = pltpu.CompilerParams()
    fields = pltpu.CompilerParams.__dataclass_fields__
    if "needs_layout_passes" in fields:
        cp = dataclasses.replace(cp, needs_layout_passes=False)
    if "use_tc_tiling_on_sc" in fields:
        cp = dataclasses.replace(cp, use_tc_tiling_on_sc=False)
    return cp


def _sc_spmm3(xall, src_cat, dst_cat, val_cat):
    mesh = plsc.VectorSubcoreMesh(
        core_axis_name="c", subcore_axis_name="s",
        num_cores=NC, num_subcores=NS)
    fn = pl.kernel(
        _sc_spmm_body,
        out_type=jax.ShapeDtypeStruct((NC * C, NPAD, HALF), _BF16),
        mesh=mesh,
        scratch_types=[
            pltpu.VMEM_SHARED((NPAD, HALF), _BF16),
            pltpu.VMEM((SB, CHUNK), jnp.int32),
            pltpu.VMEM((SB, CHUNK), jnp.int32),
            pltpu.VMEM((SB, CHUNK), _F32),
            pltpu.VMEM((2 * G, CHUNK, HALF), _BF16),
            pltpu.VMEM((ZROWS, HALF), _BF16),
            pltpu.SemaphoreType.DMA((2,)),
            pltpu.SemaphoreType.DMA((2,)),
            pltpu.SemaphoreType.DMA,
        ],
        name="sc_spmm3",
        compiler_params=_sc_compiler_params(),
    )
    return fn(xall, src_cat, dst_cat, val_cat)


_BLK = 2000
_GRID = N // _BLK


def _attention_mix(g, s1v, s2v):
    t = [jnp.tanh(jnp.dot(g[cc], s1v, preferred_element_type=_F32))
         for cc in range(C)]
    outs = []
    for i in range(C):
        s2row = s2v[i][None, :]
        l = [jnp.sum(t[cc][:, 32 * i:32 * i + 32] * s2row,
                     axis=1, keepdims=True) for cc in range(C)]
        m = jnp.maximum(jnp.maximum(l[0], l[1]), l[2])
        w = [jnp.exp(x - m) for x in l]
        tot = w[0] + w[1] + w[2]
        outs.append(
            _leaky((w[0] * g[0] + w[1] * g[1] + w[2] * g[2]) / tot))
    return outs


def _gcn_transform(y3, wgv, wgcv, cev):
    g = []
    for cc in range(C):
        e = jnp.concatenate([y3[cc], y3[C + cc]], axis=1).astype(_F32)
        h = jnp.dot(e, wgv, preferred_element_type=_F32)
        h = h * cev[cc][None, :]
        h = _leaky(jnp.dot(h, wgcv, preferred_element_type=_F32))
        g.append(h)
    return g


def _make_tc_layer(final):
    def body(*refs):
        if final:
            y3, x1, p00, p01, p02, wg, wgc, ce, s1c, s2c = refs[:10]
            outs = refs[10:]
            p0 = (p00, p01, p02)
        else:
            y3, wg, wgc, ce, s1c, s2c, wr0, wr1 = refs[:8]
            outs = refs[8:]

        g = _gcn_transform(y3, wg[...], wgc[...], ce[...])
        mix = _attention_mix(g, s1c[...], s2c[...])

        for i in range(C):
            if final:
                p1 = jnp.concatenate(
                    [x1[i], x1[C + i]], axis=1).astype(_F32)
                outs[i][...] = (p0[i][...] + p1 + mix[i]) * (1.0 / 3.0)
            else:
                ob = mix[i].astype(_BF16)
                outs[0][i] = ob[:, :HALF]
                outs[0][C + i] = ob[:, HALF:]

        if not final:
            @pl.when(pl.program_id(0) == 0)
            def _():
                c0 = ce[...].astype(_F32)
                c1 = _leaky(jnp.dot(c0, wr0[...],
                                    preferred_element_type=_F32))
                c2 = _leaky(jnp.dot(c1, wr1[...],
                                    preferred_element_type=_F32))
                outs[1][...] = c1
                outs[2][...] = (c0 + c1 + c2) * (1.0 / 3.0)

    row_spec = lambda w: pl.BlockSpec((_BLK, w), lambda b: (b, 0))
    wt_spec = lambda a, b_: pl.BlockSpec((a, b_), lambda b: (0, 0))
    y3_spec = pl.BlockSpec((2 * C, _BLK, HALF), lambda b: (0, b, 0))

    wt_specs = [wt_spec(EMB, EMB), wt_spec(EMB, EMB),
                wt_spec(8, EMB), wt_spec(EMB, 96), wt_spec(8, 32)]
    if final:
        in_specs = [y3_spec, y3_spec] + [row_spec(EMB)] * 3 + wt_specs
        out_specs = [row_spec(EMB)] * 3
        out_shape = [jax.ShapeDtypeStruct((N, EMB), _F32)] * 3
    else:
        in_specs = [y3_spec] + wt_specs + [wt_spec(EMB, EMB)] * 2
        out_specs = [
            pl.BlockSpec((2 * C, _BLK, HALF), lambda b: (0, b, 0)),
            wt_spec(8, EMB), wt_spec(8, EMB)]
        out_shape = [
            jax.ShapeDtypeStruct((2 * C, N, HALF), _BF16),
            jax.ShapeDtypeStruct((8, EMB), _F32),
            jax.ShapeDtypeStruct((8, EMB), _F32)]

    return pl.pallas_call(
        body,
        grid=(_GRID,),
        in_specs=in_specs,
        out_specs=out_specs,
        out_shape=out_shape,
    )


def kernel(adj_idx_0, adj_val_0, adj_idx_1, adj_val_1, adj_idx_2, adj_val_2,
           user_embedding, item_embedding, criterion_embedding, w_gcn,
           W_gc_0, W_gc_1, W_rel_0, W_rel_1, trans_s1, trans_s2):
    pad = EPAD - E
    adj = ((adj_idx_0, adj_val_0), (adj_idx_1, adj_val_1),
           (adj_idx_2, adj_val_2))
    src_cat = jnp.concatenate(
        [jnp.pad(ai[1], (0, pad)).reshape(EROWS, CHUNK)
         for ai, _ in adj], axis=0)
    dst_cat = jnp.concatenate(
        [jnp.pad(ai[0], (0, pad)).reshape(EROWS, CHUNK)
         for ai, _ in adj], axis=0)
    val_cat = jnp.concatenate(
        [jnp.pad(av, (0, pad)).reshape(EROWS, CHUNK)
         for _, av in adj], axis=0)

    pre = jnp.concatenate([user_embedding, item_embedding], axis=0)
    p0 = [pre[:, i, :] for i in range(C)]
    xall0 = jnp.stack(
        [p0[i][:, :HALF] for i in range(C)]
        + [p0[i][:, HALF:] for i in range(C)], axis=0).astype(_BF16)

    ce0 = jnp.pad(criterion_embedding, ((0, 8 - C), (0, 0)))
    s1c = jnp.concatenate(
        [trans_s1[i] for i in range(C)], axis=1)
    s2c = jnp.pad(jnp.squeeze(trans_s2, -1), ((0, 8 - C), (0, 0)))

    layer1 = _make_tc_layer(final=False)
    layer2 = _make_tc_layer(final=True)

    y1 = _sc_spmm3(xall0, src_cat, dst_cat, val_cat)
    x1, c1p, cmp_ = layer1(y1, w_gcn, W_gc_0, ce0, s1c, s2c,
                           W_rel_0, W_rel_1)

    y2 = _sc_spmm3(x1, src_cat, dst_cat, val_cat)
    accs = layer2(y2, x1, p0[0], p0[1], p0[2],
                  w_gcn, W_gc_1, c1p, s1c, s2c)

    acc = jnp.stack(accs, axis=1)
    users = acc[:N_USERS]
    items = jnp.concatenate(
        [acc[N_USERS:], jnp.zeros((1, C, EMB), _F32)], axis=0)
    cris = tuple(cmp_[i:i + 1] for i in range(C))
    return (users, items) + cris
